# Initial kernel scaffold; baseline (speedup 1.0000x reference)
import jax
import jax.numpy as jnp
from jax.experimental import pallas as pl


def kernel(x, edge_index, u1, c1, W1, b1, u2, c2, W2, b2):
    raise NotImplementedError("write your pallas kernel here")


# speedup vs baseline: 5.7941x; 5.7941x over previous
import functools

import jax
import jax.numpy as jnp
from jax import lax
from jax.experimental import pallas as pl
from jax.experimental.pallas import tpu as pltpu
from jax.experimental.pallas import tpu_sc as plsc

NC, NS, LANES = 2, 16, 16
CHUNK = 80
ZR = 128


def _mm_body(x_ref, w_ref, o_ref):
    o_ref[...] = jnp.dot(x_ref[...], w_ref[...],
                         preferred_element_type=jnp.float32)


def _precompute(x, wcat):
    n, d = x.shape
    dk = wcat.shape[1]
    rb = 1000
    return pl.pallas_call(
        _mm_body,
        grid=(n // rb,),
        in_specs=[pl.BlockSpec((rb, d), lambda i: (i, 0)),
                  pl.BlockSpec((d, dk), lambda i: (0, 0))],
        out_specs=pl.BlockSpec((rb, dk), lambda i: (i, 0)),
        out_shape=jax.ShapeDtypeStruct((n, dk), jnp.float32),
    )(x, wcat)


def _mesh():
    return plsc.VectorSubcoreMesh(core_axis_name="c", subcore_axis_name="s",
                                  num_cores=NC, num_subcores=NS)


def _npad(n):
    return ((n + NS * ZR - 1) // (NS * ZR)) * (NS * ZR)


def _count_pass(dst, zeros_blk, npad):
    e = dst.shape[0]
    per_w = e // (NC * NS)
    n_chunks = per_w // CHUNK
    stripe = npad // NS
    nz = stripe // ZR

    @functools.partial(
        pl.kernel,
        out_type=jax.ShapeDtypeStruct((NC, npad, 128), jnp.float32),
        mesh=_mesh(),
        compiler_params=pltpu.CompilerParams(needs_layout_passes=False),
        scratch_types=[
            pltpu.VMEM((CHUNK,), jnp.int32),
            pltpu.VMEM((CHUNK, 128), jnp.float32),
            pltpu.VMEM_SHARED((npad, 128), jnp.float32),
        ],
    )
    def k(dst_hbm, z_hbm, out_hbm, id_v, ones_v, cnt_sh):
        c = lax.axis_index("c")
        s = lax.axis_index("s")
        ebase = (c * NS + s) * per_w
        rbase = s * stripe

        @pl.loop(0, CHUNK)
        def _(kk):
            for j in range(8):
                ones_v[kk, pl.ds(j * LANES, LANES)] = jnp.full(
                    (LANES,), 1.0, jnp.float32)

        @pl.loop(0, nz)
        def _(i):
            r0 = pl.multiple_of(rbase + i * ZR, 8)
            pltpu.sync_copy(z_hbm, cnt_sh.at[pl.ds(r0, ZR)])
        plsc.subcore_barrier()

        @pl.loop(0, n_chunks)
        def _(t):
            pltpu.sync_copy(dst_hbm.at[pl.ds(ebase + t * CHUNK, CHUNK)], id_v)
            pltpu.sync_copy(ones_v, cnt_sh.at[id_v], add=True)

        plsc.subcore_barrier()

        @pl.loop(0, nz)
        def _(i):
            r0 = pl.multiple_of(rbase + i * ZR, 8)
            pltpu.sync_copy(cnt_sh.at[pl.ds(r0, ZR)],
                            out_hbm.at[c, pl.ds(r0, ZR)])

    return k(dst, zeros_blk)


def _edge_pass(ab, xv, src, dst, cc16, zeros_blk, npad):
    n = ab.shape[0]
    e = src.shape[0]
    per_w = e // (NC * NS)
    n_chunks = per_w // CHUNK
    stripe = npad // NS
    nz = stripe // ZR

    @functools.partial(
        pl.kernel,
        out_type=jax.ShapeDtypeStruct((NC, npad, 128), jnp.float32),
        mesh=_mesh(),
        compiler_params=pltpu.CompilerParams(needs_layout_passes=False),
        scratch_types=[
            pltpu.VMEM((n,), jnp.float32),
            pltpu.VMEM((LANES,), jnp.float32),
            pltpu.VMEM((CHUNK,), jnp.int32),
            pltpu.VMEM((CHUNK,), jnp.int32),
            pltpu.VMEM((CHUNK, 256), jnp.float32),
            pltpu.VMEM((CHUNK, 128), jnp.float32),
            pltpu.VMEM((CHUNK,), jnp.float32),
            pltpu.VMEM_SHARED((npad, 128), jnp.float32),
            pltpu.SemaphoreType.DMA,
        ],
    )
    def k(ab_hbm, xv_hbm, src_hbm, dst_hbm, cc_hbm, z_hbm, out_hbm,
          xv_v, cc_v, is_v, id_v, rows_v, msg_v, q_v, agg_sh, sem):
        c = lax.axis_index("c")
        s = lax.axis_index("s")
        ebase = (c * NS + s) * per_w
        rbase = s * stripe

        pltpu.sync_copy(xv_hbm, xv_v)
        pltpu.sync_copy(cc_hbm, cc_v)

        @pl.loop(0, nz)
        def _(i):
            r0 = pl.multiple_of(rbase + i * ZR, 8)
            pltpu.sync_copy(z_hbm, agg_sh.at[pl.ds(r0, ZR)])
        plsc.subcore_barrier()

        ccv = cc_v[...]

        @pl.loop(0, n_chunks)
        def _(t):
            eb = ebase + t * CHUNK
            pltpu.sync_copy(src_hbm.at[pl.ds(eb, CHUNK)], is_v)
            pltpu.sync_copy(dst_hbm.at[pl.ds(eb, CHUNK)], id_v)
            pltpu.async_copy(ab_hbm.at[is_v], rows_v, sem).wait()
            for gg in range(CHUNK // LANES):
                sl = pl.ds(gg * LANES, LANES)
                xs = plsc.load_gather(xv_v, [is_v[sl]])
                xd = plsc.load_gather(xv_v, [id_v[sl]])
                z = xs - xd + ccv
                q_v[sl] = 1.0 / (1.0 + jnp.exp(-z))

            @pl.loop(0, CHUNK)
            def _(kk):
                qs = plsc.load_gather(q_v, [jnp.full((LANES,), kk, jnp.int32)])
                for j in range(8):
                    sl = pl.ds(j * LANES, LANES)
                    slb = pl.ds(128 + j * LANES, LANES)
                    msg_v[kk, sl] = rows_v[kk, slb] + qs * rows_v[kk, sl]

            pltpu.sync_copy(msg_v, agg_sh.at[id_v], add=True)

        plsc.subcore_barrier()

        @pl.loop(0, nz)
        def _(i):
            r0 = pl.multiple_of(rbase + i * ZR, 8)
            pltpu.sync_copy(agg_sh.at[pl.ds(r0, ZR)],
                            out_hbm.at[c, pl.ds(r0, ZR)])

    return k(ab, xv, src, dst, cc16, zeros_blk)


def _post_body(p0_ref, p1_ref, c0_ref, c1_ref, ab_ref, x_ref, b_ref, s_ref,
               o_ref, *, residual):
    s0 = s_ref[0]
    agg = (p0_ref[...] + p1_ref[...]
           + ab_ref[:, 128:] + s0 * ab_ref[:, :128])
    cnt = c0_ref[:, :1] + c1_ref[:, :1] + 1.0
    h = agg / cnt + b_ref[...]
    if residual:
        h = h + x_ref[...]
    o_ref[...] = jnp.maximum(h, 0.0)


def _finish(p0, p1, c0, c1, ab, xres, b2d, s0arr, residual):
    n = ab.shape[0]
    rb = 1000
    return pl.pallas_call(
        functools.partial(_post_body, residual=residual),
        grid=(n // rb,),
        in_specs=[
            pl.BlockSpec((rb, 128), lambda i: (i, 0)),
            pl.BlockSpec((rb, 128), lambda i: (i, 0)),
            pl.BlockSpec((rb, 128), lambda i: (i, 0)),
            pl.BlockSpec((rb, 128), lambda i: (i, 0)),
            pl.BlockSpec((rb, 256), lambda i: (i, 0)),
            pl.BlockSpec((rb, 128), lambda i: (i, 0)),
            pl.BlockSpec((1, 128), lambda i: (0, 0)),
            pl.BlockSpec(memory_space=pltpu.SMEM),
        ],
        out_specs=pl.BlockSpec((rb, 128), lambda i: (i, 0)),
        out_shape=jax.ShapeDtypeStruct((n, 128), jnp.float32),
    )(p0, p1, c0, c1, ab, xres, b2d, s0arr)


def kernel(x, edge_index, u1, c1, W1, b1, u2, c2, W2, b2):
    n, d = x.shape
    src = edge_index[0]
    dst = edge_index[1]
    npad = _npad(n)
    zeros_blk = jnp.zeros((ZR, 128), jnp.float32)
    cnt = _count_pass(dst, zeros_blk, npad)
    cnt0 = cnt[0, :n]
    cnt1 = cnt[1, :n]

    def conv(xin, u, c, W, b, xres, residual):
        wa = W[:, :d] - W[:, d:]
        wb = W[:, d:]
        uv = (u[:, 0] - u[:, 1])[:, None]
        wcat = jnp.concatenate(
            [wa, wb, jnp.broadcast_to(uv, (d, 128))], axis=1)
        pre = _precompute(xin, wcat)
        ab = pre[:, :256]
        xv = pre[:, 256]
        cc = c[0] - c[1]
        cc16 = jnp.full((16,), cc, jnp.float32)
        aggcnt = _edge_pass(ab, xv, src, dst, cc16, zeros_blk, npad)
        s0arr = jax.nn.sigmoid(cc)[None]
        return _finish(aggcnt[0, :n], aggcnt[1, :n], cnt0, cnt1, ab, xres,
                       b.reshape(1, d), s0arr, residual)

    h1 = conv(x, u1, c1, W1, b1, x, residual=False)
    out = conv(h1, u2, c2, W2, b2, x, residual=True)
    return (out, edge_index)


# speedup vs baseline: 8.1337x; 1.4038x over previous
import functools

import jax
import jax.numpy as jnp
from jax import lax
from jax.experimental import pallas as pl
from jax.experimental.pallas import tpu as pltpu
from jax.experimental.pallas import tpu_sc as plsc

NC, NS, LANES = 2, 16, 16
CHUNK = 40
ZR = 32


def _mm_body(x_ref, w_ref, o_ref):
    o_ref[...] = jnp.dot(x_ref[...], w_ref[...],
                         preferred_element_type=jnp.float32)


def _precompute(x, wcat):
    n, d = x.shape
    dk = wcat.shape[1]
    rb = 1000
    return pl.pallas_call(
        _mm_body,
        grid=(n // rb,),
        in_specs=[pl.BlockSpec((rb, d), lambda i: (i, 0)),
                  pl.BlockSpec((d, dk), lambda i: (0, 0))],
        out_specs=pl.BlockSpec((rb, dk), lambda i: (i, 0)),
        out_shape=jax.ShapeDtypeStruct((n, dk), jnp.float32),
    )(x, wcat)


def _mesh():
    return plsc.VectorSubcoreMesh(core_axis_name="c", subcore_axis_name="s",
                                  num_cores=NC, num_subcores=NS)


def _npad(n):
    return ((n + NS * ZR - 1) // (NS * ZR)) * (NS * ZR)


def _edge_pass(ab, xv, ei_c, cc16, npad, with_counts):
    n = ab.shape[0]
    n_chunks_g = ei_c.shape[0]
    n_chunks = n_chunks_g // (NC * NS)
    stripe = npad // NS
    nz = stripe // ZR
    qoffs = [0, 16, 24]

    @functools.partial(
        pl.kernel,
        out_type=([jax.ShapeDtypeStruct((NC, npad, 128), jnp.float32)] * 2
                  if with_counts else
                  jax.ShapeDtypeStruct((NC, npad, 128), jnp.float32)),
        mesh=_mesh(),
        compiler_params=pltpu.CompilerParams(needs_layout_passes=False),
        scratch_types=[
            pltpu.VMEM((n,), jnp.float32),
            pltpu.VMEM((LANES,), jnp.float32),
            pltpu.VMEM((4, 2, CHUNK), jnp.int32),
            pltpu.VMEM((2, CHUNK, 256), jnp.float32),
            pltpu.VMEM((2, CHUNK, 128), jnp.float32),
            pltpu.VMEM((ZR, 128), jnp.float32),
            pltpu.VMEM((CHUNK,), jnp.float32),
            pltpu.VMEM_SHARED((npad, 128), jnp.float32),
            pltpu.SemaphoreType.DMA,
            pltpu.SemaphoreType.DMA,
            pltpu.SemaphoreType.DMA,
            pltpu.SemaphoreType.DMA,
        ],
    )
    def k(ab_hbm, xv_hbm, ei_hbm, cc_hbm, *rest):
        if with_counts:
            (out_hbm, cnt_hbm, xv_v, cc_v, idx_v, rows_v, msg_v, zb_v, q_v,
             agg_sh, gsem, ssem, isem0, isem1) = rest
        else:
            (out_hbm, xv_v, cc_v, idx_v, rows_v, msg_v, zb_v, q_v,
             agg_sh, gsem, ssem, isem0, isem1) = rest
        c = lax.axis_index("c")
        s = lax.axis_index("s")
        jbase = (c * NS + s) * n_chunks
        rbase = s * stripe

        pltpu.sync_copy(xv_hbm, xv_v)
        pltpu.sync_copy(cc_hbm, cc_v)

        @pl.loop(0, ZR)
        def _(kk):
            for j in range(8):
                zb_v[kk, pl.ds(j * LANES, LANES)] = jnp.full(
                    (LANES,), 0.0, jnp.float32)

        @pl.loop(0, nz)
        def _(i):
            r0 = pl.multiple_of(rbase + i * ZR, 8)
            pltpu.sync_copy(zb_v, agg_sh.at[pl.ds(r0, ZR)])
        plsc.subcore_barrier()

        ccv = cc_v[...]

        def wait_idx(t1):
            @pl.when(lax.rem(t1, 2) == 0)
            def _():
                pltpu.make_async_copy(ei_hbm.at[jbase + t1],
                                      idx_v.at[lax.rem(t1, 4)], isem0).wait()

            @pl.when(lax.rem(t1, 2) == 1)
            def _():
                pltpu.make_async_copy(ei_hbm.at[jbase + t1],
                                      idx_v.at[lax.rem(t1, 4)], isem1).wait()

        def issue_idx(t2):
            @pl.when(lax.rem(t2, 2) == 0)
            def _():
                pltpu.async_copy(ei_hbm.at[jbase + t2],
                                 idx_v.at[lax.rem(t2, 4)], isem0)

            @pl.when(lax.rem(t2, 2) == 1)
            def _():
                pltpu.async_copy(ei_hbm.at[jbase + t2],
                                 idx_v.at[lax.rem(t2, 4)], isem1)

        pltpu.sync_copy(ei_hbm.at[jbase], idx_v.at[0])
        pltpu.async_copy(ab_hbm.at[idx_v.at[0, 0]], rows_v.at[0], gsem)
        pltpu.async_copy(ei_hbm.at[jbase + 1], idx_v.at[1], isem1)

        @pl.loop(0, n_chunks)
        def _(t):
            p = lax.rem(t, 2)
            np_ = 1 - p
            ib = lax.rem(t, 4)
            ib1 = lax.rem(t + 1, 4)

            @pl.when(t + 1 < n_chunks)
            def _():
                wait_idx(t + 1)

            pltpu.make_async_copy(
                ab_hbm.at[idx_v.at[ib, 0]], rows_v.at[p], gsem).wait()

            @pl.when(t + 1 < n_chunks)
            def _():
                pltpu.async_copy(
                    ab_hbm.at[idx_v.at[ib1, 0]], rows_v.at[np_], gsem)

            @pl.when(t + 2 < n_chunks)
            def _():
                issue_idx(t + 2)

            for off in qoffs:
                sl = pl.ds(off, LANES)
                xs = plsc.load_gather(xv_v, [idx_v[ib, 0, sl]])
                xd = plsc.load_gather(xv_v, [idx_v[ib, 1, sl]])
                z = xs - xd + ccv
                q_v[sl] = 1.0 / (1.0 + jnp.exp(-z))

            @pl.when(t >= 1)
            def _():
                pltpu.make_async_copy(
                    msg_v.at[p], agg_sh.at[idx_v.at[ib, 1]], ssem).wait()

            @pl.loop(0, CHUNK, unroll=2)
            def _(kk):
                qs = plsc.load_gather(q_v, [jnp.full((LANES,), kk, jnp.int32)])
                for j in range(8):
                    sl = pl.ds(j * LANES, LANES)
                    slb = pl.ds(128 + j * LANES, LANES)
                    msg_v[p, kk, sl] = rows_v[p, kk, slb] + qs * rows_v[p, kk, sl]

            pltpu.async_copy(msg_v.at[p], agg_sh.at[idx_v.at[ib, 1]], ssem,
                             add=True)

        lastp = lax.rem(n_chunks - 1, 2)
        lastb = lax.rem(n_chunks - 1, 4)
        pltpu.make_async_copy(
            msg_v.at[lastp], agg_sh.at[idx_v.at[lastb, 1]], ssem).wait()
        plsc.subcore_barrier()

        @pl.loop(0, nz)
        def _(i):
            r0 = pl.multiple_of(rbase + i * ZR, 8)
            pltpu.sync_copy(agg_sh.at[pl.ds(r0, ZR)],
                            out_hbm.at[c, pl.ds(r0, ZR)])
            if with_counts:
                pltpu.sync_copy(zb_v, agg_sh.at[pl.ds(r0, ZR)])

        if with_counts:
            @pl.loop(0, CHUNK)
            def _(kk):
                for j in range(8):
                    msg_v[0, kk, pl.ds(j * LANES, LANES)] = jnp.full(
                        (LANES,), 1.0, jnp.float32)
            plsc.subcore_barrier()

            ones_ref = msg_v.at[0]
            pltpu.sync_copy(ei_hbm.at[jbase], idx_v.at[0])
            pltpu.async_copy(ei_hbm.at[jbase + 1], idx_v.at[1], isem1)

            @pl.loop(0, n_chunks)
            def _(t):
                ib = lax.rem(t, 4)

                @pl.when(t + 1 < n_chunks)
                def _():
                    wait_idx(t + 1)

                @pl.when(t + 2 < n_chunks)
                def _():
                    issue_idx(t + 2)

                @pl.when(t >= 1)
                def _():
                    pltpu.make_async_copy(
                        ones_ref, agg_sh.at[idx_v.at[ib, 1]], ssem).wait()
                pltpu.async_copy(ones_ref, agg_sh.at[idx_v.at[ib, 1]], ssem,
                                 add=True)

            lastb2 = lax.rem(n_chunks - 1, 4)
            pltpu.make_async_copy(
                ones_ref, agg_sh.at[idx_v.at[lastb2, 1]], ssem).wait()
            plsc.subcore_barrier()

            @pl.loop(0, nz)
            def _(i):
                r0 = pl.multiple_of(rbase + i * ZR, 8)
                pltpu.sync_copy(agg_sh.at[pl.ds(r0, ZR)],
                                cnt_hbm.at[c, pl.ds(r0, ZR)])

    return k(ab, xv, ei_c, cc16)


def _post_body(p0_ref, p1_ref, c0_ref, c1_ref, ab_ref, x_ref, b_ref, s_ref,
               o_ref, *, residual):
    s0 = s_ref[0]
    agg = (p0_ref[...] + p1_ref[...]
           + ab_ref[:, 128:] + s0 * ab_ref[:, :128])
    cnt = c0_ref[:, :1] + c1_ref[:, :1] + 1.0
    h = agg / cnt + b_ref[...]
    if residual:
        h = h + x_ref[...]
    o_ref[...] = jnp.maximum(h, 0.0)


def _finish(p0, p1, c0, c1, ab, xres, b2d, s0arr, residual):
    n = ab.shape[0]
    rb = 1000
    return pl.pallas_call(
        functools.partial(_post_body, residual=residual),
        grid=(n // rb,),
        in_specs=[
            pl.BlockSpec((rb, 128), lambda i: (i, 0)),
            pl.BlockSpec((rb, 128), lambda i: (i, 0)),
            pl.BlockSpec((rb, 128), lambda i: (i, 0)),
            pl.BlockSpec((rb, 128), lambda i: (i, 0)),
            pl.BlockSpec((rb, 256), lambda i: (i, 0)),
            pl.BlockSpec((rb, 128), lambda i: (i, 0)),
            pl.BlockSpec((1, 128), lambda i: (0, 0)),
            pl.BlockSpec(memory_space=pltpu.SMEM),
        ],
        out_specs=pl.BlockSpec((rb, 128), lambda i: (i, 0)),
        out_shape=jax.ShapeDtypeStruct((n, 128), jnp.float32),
    )(p0, p1, c0, c1, ab, xres, b2d, s0arr)


def kernel(x, edge_index, u1, c1, W1, b1, u2, c2, W2, b2):
    n, d = x.shape
    e = edge_index.shape[1]
    ei_c = edge_index.reshape(2, e // CHUNK, CHUNK).transpose(1, 0, 2)
    npad = _npad(n)
    cnt_box = []

    def conv(xin, u, c, W, b, xres, residual):
        wa = W[:, :d] - W[:, d:]
        wb = W[:, d:]
        uv = (u[:, 0] - u[:, 1])[:, None]
        wcat = jnp.concatenate(
            [wa, wb, jnp.broadcast_to(uv, (d, 128))], axis=1)
        pre = _precompute(xin, wcat)
        ab = pre[:, :256]
        xv = pre[:, 256]
        cc = c[0] - c[1]
        cc16 = jnp.full((16,), cc, jnp.float32)
        res = _edge_pass(ab, xv, ei_c, cc16, npad,
                         with_counts=not cnt_box)
        if not cnt_box:
            aggcnt, cntout = res
            cnt_box.append((cntout[0, :n], cntout[1, :n]))
        else:
            aggcnt = res
        cnt0, cnt1 = cnt_box[0]
        s0arr = jax.nn.sigmoid(cc)[None]
        return _finish(aggcnt[0, :n], aggcnt[1, :n], cnt0, cnt1, ab, xres,
                       b.reshape(1, d), s0arr, residual)

    h1 = conv(x, u1, c1, W1, b1, x, residual=False)
    out = conv(h1, u2, c2, W2, b2, x, residual=True)
    return (out, edge_index)


# speedup vs baseline: 8.3800x; 1.0303x over previous
import functools

import jax
import jax.numpy as jnp
from jax import lax
from jax.experimental import pallas as pl
from jax.experimental.pallas import tpu as pltpu
from jax.experimental.pallas import tpu_sc as plsc

NC, NS, LANES = 2, 16, 16
CHUNK = 40
ZR = 32


def _mm_body(x_ref, w_ref, o_ref):
    o_ref[...] = jnp.dot(x_ref[...], w_ref[...],
                         preferred_element_type=jnp.float32)


def _precompute(x, wcat):
    n, d = x.shape
    dk = wcat.shape[1]
    rb = 1000
    return pl.pallas_call(
        _mm_body,
        grid=(n // rb,),
        in_specs=[pl.BlockSpec((rb, d), lambda i: (i, 0)),
                  pl.BlockSpec((d, dk), lambda i: (0, 0))],
        out_specs=pl.BlockSpec((rb, dk), lambda i: (i, 0)),
        out_shape=jax.ShapeDtypeStruct((n, dk), jnp.float32),
    )(x, wcat)


def _mesh():
    return plsc.VectorSubcoreMesh(core_axis_name="c", subcore_axis_name="s",
                                  num_cores=NC, num_subcores=NS)


def _npad(n):
    return ((n + NS * ZR - 1) // (NS * ZR)) * (NS * ZR)


def _edge_pass(ab, xv, ei_c, dst_c, cc16, npad, with_counts):
    n = ab.shape[0]
    n_chunks_g = ei_c.shape[0]
    n_chunks = n_chunks_g // (NC * NS)
    stripe = npad // NS
    nz = stripe // ZR
    qoffs = [0, 16, 24]

    @functools.partial(
        pl.kernel,
        out_type=([jax.ShapeDtypeStruct((NC, npad, 128), jnp.float32)] * 2
                  if with_counts else
                  jax.ShapeDtypeStruct((NC, npad, 128), jnp.float32)),
        mesh=_mesh(),
        compiler_params=pltpu.CompilerParams(needs_layout_passes=False),
        scratch_types=[
            pltpu.VMEM((n,), jnp.float32),
            pltpu.VMEM((LANES,), jnp.float32),
            pltpu.VMEM((4, 2, CHUNK), jnp.int32),
            pltpu.VMEM((2, CHUNK, 256), jnp.float32),
            pltpu.VMEM((2 * CHUNK, 128), jnp.float32),
            pltpu.VMEM((4, 2 * CHUNK), jnp.int32),
            pltpu.VMEM((ZR, 128), jnp.float32),
            pltpu.VMEM((CHUNK,), jnp.float32),
            pltpu.VMEM_SHARED((npad, 128), jnp.float32),
            pltpu.SemaphoreType.DMA,
            pltpu.SemaphoreType.DMA,
            pltpu.SemaphoreType.DMA,
            pltpu.SemaphoreType.DMA,
        ],
    )
    def k(ab_hbm, xv_hbm, ei_hbm, dc_hbm, cc_hbm, *rest):
        if with_counts:
            (out_hbm, cnt_hbm, xv_v, cc_v, idx_v, rows_v, msg_v, ix2_v,
             zb_v, q_v, agg_sh, gsem, ssem, isem0, isem1) = rest
        else:
            (out_hbm, xv_v, cc_v, idx_v, rows_v, msg_v, ix2_v,
             zb_v, q_v, agg_sh, gsem, ssem, isem0, isem1) = rest
        c = lax.axis_index("c")
        s = lax.axis_index("s")
        jbase = (c * NS + s) * n_chunks
        rbase = s * stripe

        pltpu.sync_copy(xv_hbm, xv_v)
        pltpu.sync_copy(cc_hbm, cc_v)

        @pl.loop(0, ZR)
        def _(kk):
            for j in range(8):
                zb_v[kk, pl.ds(j * LANES, LANES)] = jnp.full(
                    (LANES,), 0.0, jnp.float32)

        @pl.loop(0, nz)
        def _(i):
            r0 = pl.multiple_of(rbase + i * ZR, 8)
            pltpu.sync_copy(zb_v, agg_sh.at[pl.ds(r0, ZR)])
        plsc.subcore_barrier()

        ccv = cc_v[...]

        def wait_idx(t1):
            @pl.when(lax.rem(t1, 2) == 0)
            def _():
                pltpu.make_async_copy(ei_hbm.at[jbase + t1],
                                      idx_v.at[lax.rem(t1, 4)], isem0).wait()

            @pl.when(lax.rem(t1, 2) == 1)
            def _():
                pltpu.make_async_copy(ei_hbm.at[jbase + t1],
                                      idx_v.at[lax.rem(t1, 4)], isem1).wait()

        def issue_idx(t2):
            @pl.when(lax.rem(t2, 2) == 0)
            def _():
                pltpu.async_copy(ei_hbm.at[jbase + t2],
                                 idx_v.at[lax.rem(t2, 4)], isem0)

            @pl.when(lax.rem(t2, 2) == 1)
            def _():
                pltpu.async_copy(ei_hbm.at[jbase + t2],
                                 idx_v.at[lax.rem(t2, 4)], isem1)

        pltpu.sync_copy(ei_hbm.at[jbase], idx_v.at[0])
        pltpu.async_copy(ab_hbm.at[idx_v.at[0, 0]], rows_v.at[0], gsem)
        pltpu.async_copy(ei_hbm.at[jbase + 1], idx_v.at[1], isem1)

        @pl.loop(0, n_chunks)
        def _(t):
            p = lax.rem(t, 2)
            np_ = 1 - p
            ib = lax.rem(t, 4)
            ib1 = lax.rem(t + 1, 4)

            @pl.when(t + 1 < n_chunks)
            def _():
                wait_idx(t + 1)

            pltpu.make_async_copy(
                ab_hbm.at[idx_v.at[ib, 0]], rows_v.at[p], gsem).wait()

            @pl.when(t + 1 < n_chunks)
            def _():
                pltpu.async_copy(
                    ab_hbm.at[idx_v.at[ib1, 0]], rows_v.at[np_], gsem)

            @pl.when(t + 2 < n_chunks)
            def _():
                issue_idx(t + 2)

            for off in qoffs:
                sl = pl.ds(off, LANES)
                xs = plsc.load_gather(xv_v, [idx_v[ib, 0, sl]])
                xd = plsc.load_gather(xv_v, [idx_v[ib, 1, sl]])
                z = xs - xd + ccv
                q_v[sl] = 1.0 / (1.0 + jnp.exp(-z))

            mrow = pl.multiple_of(p * CHUNK, 8)
            mref = msg_v.at[pl.ds(mrow, CHUNK)]

            @pl.when(t >= 1)
            def _():
                pltpu.make_async_copy(
                    mref, agg_sh.at[idx_v.at[ib, 1]], ssem).wait()

            @pl.loop(0, CHUNK, unroll=4)
            def _(kk):
                qs = plsc.load_gather(q_v, [jnp.full((LANES,), kk, jnp.int32)])
                kr = p * CHUNK + kk
                for j in range(8):
                    sl = pl.ds(j * LANES, LANES)
                    slb = pl.ds(128 + j * LANES, LANES)
                    msg_v[kr, sl] = rows_v[p, kk, slb] + qs * rows_v[p, kk, sl]

            pltpu.async_copy(mref, agg_sh.at[idx_v.at[ib, 1]], ssem,
                             add=True)

        lastp = pl.multiple_of(lax.rem(n_chunks - 1, 2) * CHUNK, 8)
        lastb = lax.rem(n_chunks - 1, 4)
        pltpu.make_async_copy(
            msg_v.at[pl.ds(lastp, CHUNK)], agg_sh.at[idx_v.at[lastb, 1]],
            ssem).wait()
        plsc.subcore_barrier()

        @pl.loop(0, nz)
        def _(i):
            r0 = pl.multiple_of(rbase + i * ZR, 8)
            pltpu.sync_copy(agg_sh.at[pl.ds(r0, ZR)],
                            out_hbm.at[c, pl.ds(r0, ZR)])
            if with_counts:
                pltpu.sync_copy(zb_v, agg_sh.at[pl.ds(r0, ZR)])

        if with_counts:
            nc2 = n_chunks // 2

            @pl.loop(0, 2 * CHUNK)
            def _(kk):
                for j in range(8):
                    msg_v[kk, pl.ds(j * LANES, LANES)] = jnp.full(
                        (LANES,), 1.0, jnp.float32)
            plsc.subcore_barrier()

            jb2 = (c * NS + s) * nc2

            def wait_cidx(t1):
                @pl.when(lax.rem(t1, 2) == 0)
                def _():
                    pltpu.make_async_copy(dc_hbm.at[jb2 + t1],
                                          ix2_v.at[lax.rem(t1, 4)],
                                          isem0).wait()

                @pl.when(lax.rem(t1, 2) == 1)
                def _():
                    pltpu.make_async_copy(dc_hbm.at[jb2 + t1],
                                          ix2_v.at[lax.rem(t1, 4)],
                                          isem1).wait()

            def issue_cidx(t2):
                @pl.when(lax.rem(t2, 2) == 0)
                def _():
                    pltpu.async_copy(dc_hbm.at[jb2 + t2],
                                     ix2_v.at[lax.rem(t2, 4)], isem0)

                @pl.when(lax.rem(t2, 2) == 1)
                def _():
                    pltpu.async_copy(dc_hbm.at[jb2 + t2],
                                     ix2_v.at[lax.rem(t2, 4)], isem1)

            ones_ref = msg_v.at[pl.ds(0, 2 * CHUNK)]
            pltpu.sync_copy(dc_hbm.at[jb2], ix2_v.at[0])
            pltpu.async_copy(dc_hbm.at[jb2 + 1], ix2_v.at[1], isem1)

            @pl.loop(0, nc2)
            def _(t):
                ib = lax.rem(t, 4)

                @pl.when(t + 1 < nc2)
                def _():
                    wait_cidx(t + 1)

                @pl.when(t + 2 < nc2)
                def _():
                    issue_cidx(t + 2)

                @pl.when(t >= 1)
                def _():
                    pltpu.make_async_copy(
                        ones_ref, agg_sh.at[ix2_v.at[ib]], ssem).wait()
                pltpu.async_copy(ones_ref, agg_sh.at[ix2_v.at[ib]], ssem,
                                 add=True)

            lastb2 = lax.rem(nc2 - 1, 4)
            pltpu.make_async_copy(
                ones_ref, agg_sh.at[ix2_v.at[lastb2]], ssem).wait()
            plsc.subcore_barrier()

            @pl.loop(0, nz)
            def _(i):
                r0 = pl.multiple_of(rbase + i * ZR, 8)
                pltpu.sync_copy(agg_sh.at[pl.ds(r0, ZR)],
                                cnt_hbm.at[c, pl.ds(r0, ZR)])

    return k(ab, xv, ei_c, dst_c, cc16)


def _post_body(p0_ref, p1_ref, c0_ref, c1_ref, ab_ref, x_ref, b_ref, s_ref,
               o_ref, *, residual):
    s0 = s_ref[0]
    agg = (p0_ref[...] + p1_ref[...]
           + ab_ref[:, 128:] + s0 * ab_ref[:, :128])
    cnt = c0_ref[:, :1] + c1_ref[:, :1] + 1.0
    h = agg / cnt + b_ref[...]
    if residual:
        h = h + x_ref[...]
    o_ref[...] = jnp.maximum(h, 0.0)


def _finish(p0, p1, c0, c1, ab, xres, b2d, s0arr, residual):
    n = ab.shape[0]
    rb = 1000
    return pl.pallas_call(
        functools.partial(_post_body, residual=residual),
        grid=(n // rb,),
        in_specs=[
            pl.BlockSpec((rb, 128), lambda i: (i, 0)),
            pl.BlockSpec((rb, 128), lambda i: (i, 0)),
            pl.BlockSpec((rb, 128), lambda i: (i, 0)),
            pl.BlockSpec((rb, 128), lambda i: (i, 0)),
            pl.BlockSpec((rb, 256), lambda i: (i, 0)),
            pl.BlockSpec((rb, 128), lambda i: (i, 0)),
            pl.BlockSpec((1, 128), lambda i: (0, 0)),
            pl.BlockSpec(memory_space=pltpu.SMEM),
        ],
        out_specs=pl.BlockSpec((rb, 128), lambda i: (i, 0)),
        out_shape=jax.ShapeDtypeStruct((n, 128), jnp.float32),
    )(p0, p1, c0, c1, ab, xres, b2d, s0arr)


def kernel(x, edge_index, u1, c1, W1, b1, u2, c2, W2, b2):
    n, d = x.shape
    e = edge_index.shape[1]
    ei_c = edge_index.reshape(2, e // CHUNK, CHUNK).transpose(1, 0, 2)
    dst_c = edge_index[1].reshape(e // (2 * CHUNK), 2 * CHUNK)
    npad = _npad(n)
    cnt_box = []

    def conv(xin, u, c, W, b, xres, residual):
        wa = W[:, :d] - W[:, d:]
        wb = W[:, d:]
        uv = (u[:, 0] - u[:, 1])[:, None]
        wcat = jnp.concatenate(
            [wa, wb, jnp.broadcast_to(uv, (d, 128))], axis=1)
        pre = _precompute(xin, wcat)
        ab = pre[:, :256]
        xv = pre[:, 256]
        cc = c[0] - c[1]
        cc16 = jnp.full((16,), cc, jnp.float32)
        res = _edge_pass(ab, xv, ei_c, dst_c, cc16, npad,
                         with_counts=not cnt_box)
        if not cnt_box:
            aggcnt, cntout = res
            cnt_box.append((cntout[0, :n], cntout[1, :n]))
        else:
            aggcnt = res
        cnt0, cnt1 = cnt_box[0]
        s0arr = jax.nn.sigmoid(cc)[None]
        return _finish(aggcnt[0, :n], aggcnt[1, :n], cnt0, cnt1, ab, xres,
                       b.reshape(1, d), s0arr, residual)

    h1 = conv(x, u1, c1, W1, b1, x, residual=False)
    out = conv(h1, u2, c2, W2, b2, x, residual=True)
    return (out, edge_index)


# speedup vs baseline: 13.4688x; 1.6072x over previous
import functools

import jax
import jax.numpy as jnp
from jax import lax
from jax.experimental import pallas as pl
from jax.experimental.pallas import tpu as pltpu
from jax.experimental.pallas import tpu_sc as plsc

NC, NS, LANES = 2, 16, 16
CHUNK = 40
ZR = 32


def _mm_body(x_ref, w_ref, o_ref):
    o_ref[...] = jnp.dot(x_ref[...], w_ref[...],
                         preferred_element_type=jnp.float32)


def _precompute(x, wcat):
    n, d = x.shape
    dk = wcat.shape[1]
    rb = 1000
    return pl.pallas_call(
        _mm_body,
        grid=(n // rb,),
        in_specs=[pl.BlockSpec((rb, d), lambda i: (i, 0)),
                  pl.BlockSpec((d, dk), lambda i: (0, 0))],
        out_specs=pl.BlockSpec((rb, dk), lambda i: (i, 0)),
        out_shape=jax.ShapeDtypeStruct((n, dk), jnp.float32),
    )(x, wcat)


def _mesh():
    return plsc.VectorSubcoreMesh(core_axis_name="c", subcore_axis_name="s",
                                  num_cores=NC, num_subcores=NS)


def _npad(n):
    return ((n + NS * ZR - 1) // (NS * ZR)) * (NS * ZR)


def _edge_pass(ab, xv, ei_c, dst_c, cc16, npad, with_counts):
    n = ab.shape[0]
    n_chunks_g = ei_c.shape[0]
    n_chunks = n_chunks_g // (NC * NS)
    stripe = npad // NS
    nz = stripe // ZR
    qoffs = [0, 16, 24]

    @functools.partial(
        pl.kernel,
        out_type=([jax.ShapeDtypeStruct((NC, npad, 128), jnp.float32)] * 2
                  if with_counts else
                  jax.ShapeDtypeStruct((NC, npad, 128), jnp.float32)),
        mesh=_mesh(),
        compiler_params=pltpu.CompilerParams(needs_layout_passes=False),
        scratch_types=[
            pltpu.VMEM((n,), jnp.float32),
            pltpu.VMEM((LANES,), jnp.float32),
            pltpu.VMEM((4, 2, CHUNK), jnp.int32),
            pltpu.VMEM((2, CHUNK, 256), jnp.float32),
            pltpu.VMEM((2 * CHUNK, 128), jnp.float32),
            pltpu.VMEM((4, 2 * CHUNK), jnp.int32),
            pltpu.VMEM((ZR, 128), jnp.float32),
            pltpu.VMEM((CHUNK,), jnp.float32),
            pltpu.VMEM_SHARED((npad, 128), jnp.float32),
            pltpu.SemaphoreType.DMA,
            pltpu.SemaphoreType.DMA,
            pltpu.SemaphoreType.DMA,
            pltpu.SemaphoreType.DMA,
        ],
    )
    def k(ab_hbm, xv_hbm, ei_hbm, dc_hbm, cc_hbm, *rest):
        if with_counts:
            (out_hbm, cnt_hbm, xv_v, cc_v, idx_v, rows_v, msg_v, ix2_v,
             zb_v, q_v, agg_sh, gsem, ssem, isem0, isem1) = rest
        else:
            (out_hbm, xv_v, cc_v, idx_v, rows_v, msg_v, ix2_v,
             zb_v, q_v, agg_sh, gsem, ssem, isem0, isem1) = rest
        c = lax.axis_index("c")
        s = lax.axis_index("s")
        jbase = (c * NS + s) * n_chunks
        rbase = s * stripe

        pltpu.sync_copy(xv_hbm, xv_v)
        pltpu.sync_copy(cc_hbm, cc_v)

        @pl.loop(0, ZR)
        def _(kk):
            for j in range(8):
                zb_v[kk, pl.ds(j * LANES, LANES)] = jnp.full(
                    (LANES,), 0.0, jnp.float32)

        @pl.loop(0, nz)
        def _(i):
            r0 = pl.multiple_of(rbase + i * ZR, 8)
            pltpu.sync_copy(zb_v, agg_sh.at[pl.ds(r0, ZR)])
        plsc.subcore_barrier()

        ccv = cc_v[...]

        def wait_idx(t1):
            @pl.when(lax.rem(t1, 2) == 0)
            def _():
                pltpu.make_async_copy(ei_hbm.at[jbase + t1],
                                      idx_v.at[lax.rem(t1, 4)], isem0).wait()

            @pl.when(lax.rem(t1, 2) == 1)
            def _():
                pltpu.make_async_copy(ei_hbm.at[jbase + t1],
                                      idx_v.at[lax.rem(t1, 4)], isem1).wait()

        def issue_idx(t2):
            @pl.when(lax.rem(t2, 2) == 0)
            def _():
                pltpu.async_copy(ei_hbm.at[jbase + t2],
                                 idx_v.at[lax.rem(t2, 4)], isem0)

            @pl.when(lax.rem(t2, 2) == 1)
            def _():
                pltpu.async_copy(ei_hbm.at[jbase + t2],
                                 idx_v.at[lax.rem(t2, 4)], isem1)

        pltpu.sync_copy(ei_hbm.at[jbase], idx_v.at[0])
        pltpu.async_copy(ab_hbm.at[idx_v.at[0, 0]], rows_v.at[0], gsem)
        pltpu.async_copy(ei_hbm.at[jbase + 1], idx_v.at[1], isem1)

        @pl.loop(0, n_chunks)
        def _(t):
            p = lax.rem(t, 2)
            np_ = 1 - p
            ib = lax.rem(t, 4)
            ib1 = lax.rem(t + 1, 4)

            @pl.when(t + 1 < n_chunks)
            def _():
                wait_idx(t + 1)

            pltpu.make_async_copy(
                ab_hbm.at[idx_v.at[ib, 0]], rows_v.at[p], gsem).wait()

            @pl.when(t + 1 < n_chunks)
            def _():
                pltpu.async_copy(
                    ab_hbm.at[idx_v.at[ib1, 0]], rows_v.at[np_], gsem)

            @pl.when(t + 2 < n_chunks)
            def _():
                issue_idx(t + 2)

            for off in qoffs:
                sl = pl.ds(off, LANES)
                xs = plsc.load_gather(xv_v, [idx_v[ib, 0, sl]])
                xd = plsc.load_gather(xv_v, [idx_v[ib, 1, sl]])
                z = xs - xd + ccv
                q_v[sl] = 1.0 / (1.0 + jnp.exp(-z))

            mrow = pl.multiple_of(p * CHUNK, 8)
            mref = msg_v.at[pl.ds(mrow, CHUNK)]

            @pl.when(t >= 1)
            def _():
                pltpu.make_async_copy(
                    mref, agg_sh.at[idx_v.at[ib, 1]], ssem).wait()

            @plsc.parallel_loop(0, CHUNK, unroll=4)
            def _(kk):
                qs = plsc.load_gather(q_v, [jnp.full((LANES,), kk, jnp.int32)])
                kr = p * CHUNK + kk
                for j in range(8):
                    sl = pl.ds(j * LANES, LANES)
                    slb = pl.ds(128 + j * LANES, LANES)
                    msg_v[kr, sl] = rows_v[p, kk, slb] + qs * rows_v[p, kk, sl]

            pltpu.async_copy(mref, agg_sh.at[idx_v.at[ib, 1]], ssem,
                             add=True)

        lastp = pl.multiple_of(lax.rem(n_chunks - 1, 2) * CHUNK, 8)
        lastb = lax.rem(n_chunks - 1, 4)
        pltpu.make_async_copy(
            msg_v.at[pl.ds(lastp, CHUNK)], agg_sh.at[idx_v.at[lastb, 1]],
            ssem).wait()
        plsc.subcore_barrier()

        @pl.loop(0, nz)
        def _(i):
            r0 = pl.multiple_of(rbase + i * ZR, 8)
            pltpu.sync_copy(agg_sh.at[pl.ds(r0, ZR)],
                            out_hbm.at[c, pl.ds(r0, ZR)])
            if with_counts:
                pltpu.sync_copy(zb_v, agg_sh.at[pl.ds(r0, ZR)])

        if with_counts:
            nc2 = n_chunks // 2

            @pl.loop(0, 2 * CHUNK)
            def _(kk):
                for j in range(8):
                    msg_v[kk, pl.ds(j * LANES, LANES)] = jnp.full(
                        (LANES,), 1.0, jnp.float32)
            plsc.subcore_barrier()

            jb2 = (c * NS + s) * nc2

            def wait_cidx(t1):
                @pl.when(lax.rem(t1, 2) == 0)
                def _():
                    pltpu.make_async_copy(dc_hbm.at[jb2 + t1],
                                          ix2_v.at[lax.rem(t1, 4)],
                                          isem0).wait()

                @pl.when(lax.rem(t1, 2) == 1)
                def _():
                    pltpu.make_async_copy(dc_hbm.at[jb2 + t1],
                                          ix2_v.at[lax.rem(t1, 4)],
                                          isem1).wait()

            def issue_cidx(t2):
                @pl.when(lax.rem(t2, 2) == 0)
                def _():
                    pltpu.async_copy(dc_hbm.at[jb2 + t2],
                                     ix2_v.at[lax.rem(t2, 4)], isem0)

                @pl.when(lax.rem(t2, 2) == 1)
                def _():
                    pltpu.async_copy(dc_hbm.at[jb2 + t2],
                                     ix2_v.at[lax.rem(t2, 4)], isem1)

            ones_ref = msg_v.at[pl.ds(0, 2 * CHUNK)]
            pltpu.sync_copy(dc_hbm.at[jb2], ix2_v.at[0])
            pltpu.async_copy(dc_hbm.at[jb2 + 1], ix2_v.at[1], isem1)

            @pl.loop(0, nc2)
            def _(t):
                ib = lax.rem(t, 4)

                @pl.when(t + 1 < nc2)
                def _():
                    wait_cidx(t + 1)

                @pl.when(t + 2 < nc2)
                def _():
                    issue_cidx(t + 2)

                @pl.when(t >= 1)
                def _():
                    pltpu.make_async_copy(
                        ones_ref, agg_sh.at[ix2_v.at[ib]], ssem).wait()
                pltpu.async_copy(ones_ref, agg_sh.at[ix2_v.at[ib]], ssem,
                                 add=True)

            lastb2 = lax.rem(nc2 - 1, 4)
            pltpu.make_async_copy(
                ones_ref, agg_sh.at[ix2_v.at[lastb2]], ssem).wait()
            plsc.subcore_barrier()

            @pl.loop(0, nz)
            def _(i):
                r0 = pl.multiple_of(rbase + i * ZR, 8)
                pltpu.sync_copy(agg_sh.at[pl.ds(r0, ZR)],
                                cnt_hbm.at[c, pl.ds(r0, ZR)])

    return k(ab, xv, ei_c, dst_c, cc16)


def _post_body(p0_ref, p1_ref, c0_ref, c1_ref, ab_ref, x_ref, b_ref, s_ref,
               o_ref, *, residual):
    s0 = s_ref[0]
    agg = (p0_ref[...] + p1_ref[...]
           + ab_ref[:, 128:] + s0 * ab_ref[:, :128])
    cnt = c0_ref[:, :1] + c1_ref[:, :1] + 1.0
    h = agg / cnt + b_ref[...]
    if residual:
        h = h + x_ref[...]
    o_ref[...] = jnp.maximum(h, 0.0)


def _finish(p0, p1, c0, c1, ab, xres, b2d, s0arr, residual):
    n = ab.shape[0]
    rb = 1000
    return pl.pallas_call(
        functools.partial(_post_body, residual=residual),
        grid=(n // rb,),
        in_specs=[
            pl.BlockSpec((rb, 128), lambda i: (i, 0)),
            pl.BlockSpec((rb, 128), lambda i: (i, 0)),
            pl.BlockSpec((rb, 128), lambda i: (i, 0)),
            pl.BlockSpec((rb, 128), lambda i: (i, 0)),
            pl.BlockSpec((rb, 256), lambda i: (i, 0)),
            pl.BlockSpec((rb, 128), lambda i: (i, 0)),
            pl.BlockSpec((1, 128), lambda i: (0, 0)),
            pl.BlockSpec(memory_space=pltpu.SMEM),
        ],
        out_specs=pl.BlockSpec((rb, 128), lambda i: (i, 0)),
        out_shape=jax.ShapeDtypeStruct((n, 128), jnp.float32),
    )(p0, p1, c0, c1, ab, xres, b2d, s0arr)


def kernel(x, edge_index, u1, c1, W1, b1, u2, c2, W2, b2):
    n, d = x.shape
    e = edge_index.shape[1]
    ei_c = edge_index.reshape(2, e // CHUNK, CHUNK).transpose(1, 0, 2)
    dst_c = edge_index[1].reshape(e // (2 * CHUNK), 2 * CHUNK)
    npad = _npad(n)
    cnt_box = []

    def conv(xin, u, c, W, b, xres, residual):
        wa = W[:, :d] - W[:, d:]
        wb = W[:, d:]
        uv = (u[:, 0] - u[:, 1])[:, None]
        wcat = jnp.concatenate(
            [wa, wb, jnp.broadcast_to(uv, (d, 128))], axis=1)
        pre = _precompute(xin, wcat)
        ab = pre[:, :256]
        xv = pre[:, 256]
        cc = c[0] - c[1]
        cc16 = jnp.full((16,), cc, jnp.float32)
        res = _edge_pass(ab, xv, ei_c, dst_c, cc16, npad,
                         with_counts=not cnt_box)
        if not cnt_box:
            aggcnt, cntout = res
            cnt_box.append((cntout[0, :n], cntout[1, :n]))
        else:
            aggcnt = res
        cnt0, cnt1 = cnt_box[0]
        s0arr = jax.nn.sigmoid(cc)[None]
        return _finish(aggcnt[0, :n], aggcnt[1, :n], cnt0, cnt1, ab, xres,
                       b.reshape(1, d), s0arr, residual)

    h1 = conv(x, u1, c1, W1, b1, x, residual=False)
    out = conv(h1, u2, c2, W2, b2, x, residual=True)
    return (out, edge_index)


# speedup vs baseline: 13.7602x; 1.0216x over previous
import functools

import jax
import jax.numpy as jnp
from jax import lax
from jax.experimental import pallas as pl
from jax.experimental.pallas import tpu as pltpu
from jax.experimental.pallas import tpu_sc as plsc

NC, NS, LANES = 2, 16, 16
CHUNK = 40
ZR = 32


def _mm_body(x_ref, w_ref, o_ref):
    o_ref[...] = jnp.dot(x_ref[...], w_ref[...],
                         preferred_element_type=jnp.float32)


def _precompute(x, wcat):
    n, d = x.shape
    dk = wcat.shape[1]
    rb = 1000
    return pl.pallas_call(
        _mm_body,
        grid=(n // rb,),
        in_specs=[pl.BlockSpec((rb, d), lambda i: (i, 0)),
                  pl.BlockSpec((d, dk), lambda i: (0, 0))],
        out_specs=pl.BlockSpec((rb, dk), lambda i: (i, 0)),
        out_shape=jax.ShapeDtypeStruct((n, dk), jnp.float32),
    )(x, wcat)


def _mesh():
    return plsc.VectorSubcoreMesh(core_axis_name="c", subcore_axis_name="s",
                                  num_cores=NC, num_subcores=NS)


def _npad(n):
    return ((n + NS * ZR - 1) // (NS * ZR)) * (NS * ZR)


def _count_pass(dst_c, npad):
    CH2 = dst_c.shape[1]
    nc2 = dst_c.shape[0] // (NC * NS)
    stripe = npad // NS
    nz = stripe // ZR

    @functools.partial(
        pl.kernel,
        out_type=jax.ShapeDtypeStruct((NC, npad, 128), jnp.float32),
        mesh=_mesh(),
        compiler_params=pltpu.CompilerParams(needs_layout_passes=False),
        scratch_types=[
            pltpu.VMEM((4, CH2), jnp.int32),
            pltpu.VMEM((CH2, 128), jnp.float32),
            pltpu.VMEM((ZR, 128), jnp.float32),
            pltpu.VMEM_SHARED((npad, 128), jnp.float32),
            pltpu.SemaphoreType.DMA,
            pltpu.SemaphoreType.DMA,
            pltpu.SemaphoreType.DMA,
        ],
    )
    def k(dc_hbm, out_hbm, ix2_v, ones_v, zb_v, cnt_sh, ssem, isem0, isem1):
        c = lax.axis_index("c")
        s = lax.axis_index("s")
        jb2 = (c * NS + s) * nc2
        rbase = s * stripe

        @pl.loop(0, ZR)
        def _(kk):
            for j in range(8):
                zb_v[kk, pl.ds(j * LANES, LANES)] = jnp.full(
                    (LANES,), 0.0, jnp.float32)

        @pl.loop(0, CH2)
        def _(kk):
            for j in range(8):
                ones_v[kk, pl.ds(j * LANES, LANES)] = jnp.full(
                    (LANES,), 1.0, jnp.float32)

        @pl.loop(0, nz)
        def _(i):
            r0 = pl.multiple_of(rbase + i * ZR, 8)
            pltpu.sync_copy(zb_v, cnt_sh.at[pl.ds(r0, ZR)])
        plsc.subcore_barrier()

        def wait_cidx(t1):
            @pl.when(lax.rem(t1, 2) == 0)
            def _():
                pltpu.make_async_copy(dc_hbm.at[jb2 + t1],
                                      ix2_v.at[lax.rem(t1, 4)], isem0).wait()

            @pl.when(lax.rem(t1, 2) == 1)
            def _():
                pltpu.make_async_copy(dc_hbm.at[jb2 + t1],
                                      ix2_v.at[lax.rem(t1, 4)], isem1).wait()

        def issue_cidx(t2):
            @pl.when(lax.rem(t2, 2) == 0)
            def _():
                pltpu.async_copy(dc_hbm.at[jb2 + t2],
                                 ix2_v.at[lax.rem(t2, 4)], isem0)

            @pl.when(lax.rem(t2, 2) == 1)
            def _():
                pltpu.async_copy(dc_hbm.at[jb2 + t2],
                                 ix2_v.at[lax.rem(t2, 4)], isem1)

        pltpu.sync_copy(dc_hbm.at[jb2], ix2_v.at[0])
        pltpu.async_copy(dc_hbm.at[jb2 + 1], ix2_v.at[1], isem1)

        @pl.loop(0, nc2)
        def _(t):
            ib = lax.rem(t, 4)

            @pl.when(t + 1 < nc2)
            def _():
                wait_cidx(t + 1)

            @pl.when(t + 2 < nc2)
            def _():
                issue_cidx(t + 2)

            @pl.when(t >= 1)
            def _():
                pltpu.make_async_copy(
                    ones_v, cnt_sh.at[ix2_v.at[ib]], ssem).wait()
            pltpu.async_copy(ones_v, cnt_sh.at[ix2_v.at[ib]], ssem, add=True)

        lastb2 = lax.rem(nc2 - 1, 4)
        pltpu.make_async_copy(
            ones_v, cnt_sh.at[ix2_v.at[lastb2]], ssem).wait()
        plsc.subcore_barrier()

        @pl.loop(0, nz)
        def _(i):
            r0 = pl.multiple_of(rbase + i * ZR, 8)
            pltpu.sync_copy(cnt_sh.at[pl.ds(r0, ZR)],
                            out_hbm.at[c, pl.ds(r0, ZR)])

    return k(dst_c)


def _edge_pass(ab, xv, ei_c, cc16, npad):
    n = ab.shape[0]
    n_chunks_g = ei_c.shape[0]
    n_chunks = n_chunks_g // (NC * NS)
    stripe = npad // NS
    nz = stripe // ZR
    qoffs = [0, 16, 24]

    @functools.partial(
        pl.kernel,
        out_type=jax.ShapeDtypeStruct((NC, npad, 128), jnp.float32),
        mesh=_mesh(),
        compiler_params=pltpu.CompilerParams(needs_layout_passes=False),
        scratch_types=[
            pltpu.VMEM((n,), jnp.float32),
            pltpu.VMEM((LANES,), jnp.float32),
            pltpu.VMEM((4, 2, CHUNK), jnp.int32),
            pltpu.VMEM((2, CHUNK, 256), jnp.float32),
            pltpu.VMEM((2 * CHUNK, 128), jnp.float32),
            pltpu.VMEM((ZR, 128), jnp.float32),
            pltpu.VMEM((CHUNK,), jnp.float32),
            pltpu.VMEM_SHARED((npad, 128), jnp.float32),
            pltpu.SemaphoreType.DMA,
            pltpu.SemaphoreType.DMA,
            pltpu.SemaphoreType.DMA,
            pltpu.SemaphoreType.DMA,
        ],
    )
    def k(ab_hbm, xv_hbm, ei_hbm, cc_hbm, out_hbm,
          xv_v, cc_v, idx_v, rows_v, msg_v, zb_v, q_v,
          agg_sh, gsem, ssem, isem0, isem1):
        c = lax.axis_index("c")
        s = lax.axis_index("s")
        jbase = (c * NS + s) * n_chunks
        rbase = s * stripe

        pltpu.sync_copy(xv_hbm, xv_v)
        pltpu.sync_copy(cc_hbm, cc_v)

        @pl.loop(0, ZR)
        def _(kk):
            for j in range(8):
                zb_v[kk, pl.ds(j * LANES, LANES)] = jnp.full(
                    (LANES,), 0.0, jnp.float32)

        @pl.loop(0, nz)
        def _(i):
            r0 = pl.multiple_of(rbase + i * ZR, 8)
            pltpu.sync_copy(zb_v, agg_sh.at[pl.ds(r0, ZR)])
        plsc.subcore_barrier()

        ccv = cc_v[...]

        def wait_idx(t1):
            @pl.when(lax.rem(t1, 2) == 0)
            def _():
                pltpu.make_async_copy(ei_hbm.at[jbase + t1],
                                      idx_v.at[lax.rem(t1, 4)], isem0).wait()

            @pl.when(lax.rem(t1, 2) == 1)
            def _():
                pltpu.make_async_copy(ei_hbm.at[jbase + t1],
                                      idx_v.at[lax.rem(t1, 4)], isem1).wait()

        def issue_idx(t2):
            @pl.when(lax.rem(t2, 2) == 0)
            def _():
                pltpu.async_copy(ei_hbm.at[jbase + t2],
                                 idx_v.at[lax.rem(t2, 4)], isem0)

            @pl.when(lax.rem(t2, 2) == 1)
            def _():
                pltpu.async_copy(ei_hbm.at[jbase + t2],
                                 idx_v.at[lax.rem(t2, 4)], isem1)

        pltpu.sync_copy(ei_hbm.at[jbase], idx_v.at[0])
        pltpu.async_copy(ab_hbm.at[idx_v.at[0, 0]], rows_v.at[0], gsem)
        pltpu.async_copy(ei_hbm.at[jbase + 1], idx_v.at[1], isem1)

        @pl.loop(0, n_chunks)
        def _(t):
            p = lax.rem(t, 2)
            np_ = 1 - p
            ib = lax.rem(t, 4)
            ib1 = lax.rem(t + 1, 4)

            @pl.when(t + 1 < n_chunks)
            def _():
                wait_idx(t + 1)

            pltpu.make_async_copy(
                ab_hbm.at[idx_v.at[ib, 0]], rows_v.at[p], gsem).wait()

            @pl.when(t + 1 < n_chunks)
            def _():
                pltpu.async_copy(
                    ab_hbm.at[idx_v.at[ib1, 0]], rows_v.at[np_], gsem)

            @pl.when(t + 2 < n_chunks)
            def _():
                issue_idx(t + 2)

            for off in qoffs:
                sl = pl.ds(off, LANES)
                xs = plsc.load_gather(xv_v, [idx_v[ib, 0, sl]])
                xd = plsc.load_gather(xv_v, [idx_v[ib, 1, sl]])
                z = xs - xd + ccv
                q_v[sl] = 1.0 / (1.0 + jnp.exp(-z))

            mrow = pl.multiple_of(p * CHUNK, 8)
            mref = msg_v.at[pl.ds(mrow, CHUNK)]

            @pl.when(t >= 1)
            def _():
                pltpu.make_async_copy(
                    mref, agg_sh.at[idx_v.at[ib, 1]], ssem).wait()

            @plsc.parallel_loop(0, CHUNK, unroll=4)
            def _(kk):
                qs = plsc.load_gather(q_v, [jnp.full((LANES,), kk, jnp.int32)])
                kr = p * CHUNK + kk
                for j in range(8):
                    sl = pl.ds(j * LANES, LANES)
                    slb = pl.ds(128 + j * LANES, LANES)
                    msg_v[kr, sl] = rows_v[p, kk, slb] + qs * rows_v[p, kk, sl]

            pltpu.async_copy(mref, agg_sh.at[idx_v.at[ib, 1]], ssem,
                             add=True)

        lastp = pl.multiple_of(lax.rem(n_chunks - 1, 2) * CHUNK, 8)
        lastb = lax.rem(n_chunks - 1, 4)
        pltpu.make_async_copy(
            msg_v.at[pl.ds(lastp, CHUNK)], agg_sh.at[idx_v.at[lastb, 1]],
            ssem).wait()
        plsc.subcore_barrier()

        @pl.loop(0, nz)
        def _(i):
            r0 = pl.multiple_of(rbase + i * ZR, 8)
            pltpu.sync_copy(agg_sh.at[pl.ds(r0, ZR)],
                            out_hbm.at[c, pl.ds(r0, ZR)])

    return k(ab, xv, ei_c, cc16)


def _post_mm_body(p0_ref, p1_ref, c0_ref, c1_ref, ab_ref, w_ref, b_ref,
                  s_ref, o_ref):
    s0 = s_ref[0]
    agg = (p0_ref[...] + p1_ref[...]
           + ab_ref[:, 128:] + s0 * ab_ref[:, :128])
    cnt = c0_ref[:, :1] + c1_ref[:, :1] + 1.0
    h = jnp.maximum(agg / cnt + b_ref[...], 0.0)
    o_ref[...] = jnp.dot(h, w_ref[...], preferred_element_type=jnp.float32)


def _finish_mm(p0, p1, c0, c1, ab, wcat2, b2d, s0arr):
    n = ab.shape[0]
    dk = wcat2.shape[1]
    rb = 1000
    return pl.pallas_call(
        _post_mm_body,
        grid=(n // rb,),
        in_specs=[
            pl.BlockSpec((rb, 128), lambda i: (i, 0)),
            pl.BlockSpec((rb, 128), lambda i: (i, 0)),
            pl.BlockSpec((rb, 128), lambda i: (i, 0)),
            pl.BlockSpec((rb, 128), lambda i: (i, 0)),
            pl.BlockSpec((rb, 256), lambda i: (i, 0)),
            pl.BlockSpec((128, dk), lambda i: (0, 0)),
            pl.BlockSpec((1, 128), lambda i: (0, 0)),
            pl.BlockSpec(memory_space=pltpu.SMEM),
        ],
        out_specs=pl.BlockSpec((rb, dk), lambda i: (i, 0)),
        out_shape=jax.ShapeDtypeStruct((n, dk), jnp.float32),
    )(p0, p1, c0, c1, ab, wcat2, b2d, s0arr)


def _post_body(p0_ref, p1_ref, c0_ref, c1_ref, ab_ref, x_ref, b_ref, s_ref,
               o_ref, *, residual):
    s0 = s_ref[0]
    agg = (p0_ref[...] + p1_ref[...]
           + ab_ref[:, 128:] + s0 * ab_ref[:, :128])
    cnt = c0_ref[:, :1] + c1_ref[:, :1] + 1.0
    h = agg / cnt + b_ref[...]
    if residual:
        h = h + x_ref[...]
    o_ref[...] = jnp.maximum(h, 0.0)


def _finish(p0, p1, c0, c1, ab, xres, b2d, s0arr, residual):
    n = ab.shape[0]
    rb = 1000
    return pl.pallas_call(
        functools.partial(_post_body, residual=residual),
        grid=(n // rb,),
        in_specs=[
            pl.BlockSpec((rb, 128), lambda i: (i, 0)),
            pl.BlockSpec((rb, 128), lambda i: (i, 0)),
            pl.BlockSpec((rb, 128), lambda i: (i, 0)),
            pl.BlockSpec((rb, 128), lambda i: (i, 0)),
            pl.BlockSpec((rb, 256), lambda i: (i, 0)),
            pl.BlockSpec((rb, 128), lambda i: (i, 0)),
            pl.BlockSpec((1, 128), lambda i: (0, 0)),
            pl.BlockSpec(memory_space=pltpu.SMEM),
        ],
        out_specs=pl.BlockSpec((rb, 128), lambda i: (i, 0)),
        out_shape=jax.ShapeDtypeStruct((n, 128), jnp.float32),
    )(p0, p1, c0, c1, ab, xres, b2d, s0arr)


def kernel(x, edge_index, u1, c1, W1, b1, u2, c2, W2, b2):
    n, d = x.shape
    e = edge_index.shape[1]
    ei_c = edge_index.reshape(2, e // CHUNK, CHUNK).transpose(1, 0, 2)
    dst_c = edge_index[1].reshape(e // (2 * CHUNK), 2 * CHUNK)
    npad = _npad(n)

    cnt = _count_pass(dst_c, npad)
    cnt0 = cnt[0, :n]
    cnt1 = cnt[1, :n]

    def wcat_of(u, W):
        wa = W[:, :d] - W[:, d:]
        wb = W[:, d:]
        uv = (u[:, 0] - u[:, 1])[:, None]
        return jnp.concatenate(
            [wa, wb, jnp.broadcast_to(uv, (d, 128))], axis=1)

    wcat1 = wcat_of(u1, W1)
    wcat2 = wcat_of(u2, W2)
    cc1 = c1[0] - c1[1]
    cc2 = c2[0] - c2[1]

    pre1 = _precompute(x, wcat1)
    ab1 = pre1[:, :256]
    xv1 = pre1[:, 256]
    agg1 = _edge_pass(ab1, xv1, ei_c, jnp.full((16,), cc1, jnp.float32), npad)
    pre2 = _finish_mm(agg1[0, :n], agg1[1, :n], cnt0, cnt1, ab1, wcat2,
                      b1.reshape(1, d), jax.nn.sigmoid(cc1)[None])
    ab2 = pre2[:, :256]
    xv2 = pre2[:, 256]
    agg2 = _edge_pass(ab2, xv2, ei_c, jnp.full((16,), cc2, jnp.float32), npad)
    out = _finish(agg2[0, :n], agg2[1, :n], cnt0, cnt1, ab2, x,
                  b2.reshape(1, d), jax.nn.sigmoid(cc2)[None], residual=True)
    return (out, edge_index)
